# baseline (speedup 1.0000x reference)
import jax
import jax.numpy as jnp
from jax.experimental import pallas as pl
from jax.experimental.pallas import tpu as pltpu

_N, _D = 1_000_000, 3
_B = 376832
_GRID = -(-_N // _B)


def _copy_body(xt_ref, ot_ref):
    vals = xt_ref[...]

    @pl.when(pl.program_id(0) == 0)
    def _():
        r = jax.lax.broadcasted_iota(jnp.int32, (_D, _B), 0).astype(jnp.float32)
        c = jax.lax.broadcasted_iota(jnp.int32, (_D, _B), 1)
        patched = jnp.where(c == 0, 10.0 + r, jnp.where(c == 2, 20.0 + r, vals))
        ot_ref[...] = patched

    @pl.when(pl.program_id(0) != 0)
    def _():
        ot_ref[...] = vals


def kernel(x):
    xt = jnp.swapaxes(x, 0, 1)
    out_t = pl.pallas_call(
        _copy_body,
        grid=(_GRID,),
        in_specs=[pl.BlockSpec((_D, _B), lambda i: (0, i))],
        out_specs=pl.BlockSpec((_D, _B), lambda i: (0, i)),
        out_shape=jax.ShapeDtypeStruct((_D, _N), jnp.float32),
        compiler_params=pltpu.CompilerParams(vmem_limit_bytes=100 * 1024 * 1024),
    )(xt)
    return jnp.swapaxes(out_t, 0, 1)


# speedup vs baseline: 1.0318x; 1.0318x over previous
import jax
import jax.numpy as jnp
from jax.experimental import pallas as pl
from jax.experimental.pallas import tpu as pltpu

_N, _D = 1_000_000, 3
_B = 507904
_GRID = -(-_N // _B)


def _copy_body(xt_ref, ot_ref):
    vals = xt_ref[...]

    @pl.when(pl.program_id(0) == 0)
    def _():
        r = jax.lax.broadcasted_iota(jnp.int32, (_D, _B), 0).astype(jnp.float32)
        c = jax.lax.broadcasted_iota(jnp.int32, (_D, _B), 1)
        patched = jnp.where(c == 0, 10.0 + r, jnp.where(c == 2, 20.0 + r, vals))
        ot_ref[...] = patched

    @pl.when(pl.program_id(0) != 0)
    def _():
        ot_ref[...] = vals


def kernel(x):
    xt = jnp.swapaxes(x, 0, 1)
    out_t = pl.pallas_call(
        _copy_body,
        grid=(_GRID,),
        in_specs=[pl.BlockSpec((_D, _B), lambda i: (0, i))],
        out_specs=pl.BlockSpec((_D, _B), lambda i: (0, i)),
        out_shape=jax.ShapeDtypeStruct((_D, _N), jnp.float32),
        compiler_params=pltpu.CompilerParams(vmem_limit_bytes=100 * 1024 * 1024),
    )(xt)
    return jnp.swapaxes(out_t, 0, 1)
